# baseline (speedup 1.0000x reference)
import functools

import jax
import jax.numpy as jnp
from jax import lax
from jax.experimental import pallas as pl
from jax.experimental.pallas import tpu as pltpu
from jax.experimental.pallas import tpu_sc as plsc

B = 16384
K = 68
E = 67
NC = 2
NS = 16
NW = NC * NS
BT = 128
NBT = B // BT
TPW = NBT // NW
S = BT // 16
UNROLL = 4
NLOOP = 16
WEIGHT = 0.05


def _sc_body(cv, mv, out_hbm, cbuf, mbuf, accbuf, csem, msem):
    wid = lax.axis_index("s") * NC + lax.axis_index("c")
    bt0 = wid * TPW

    def ccopy(t):
        return pltpu.make_async_copy(
            cv.at[:, pl.ds(bt0 + t, 1)], cbuf.at[:, pl.ds(t, 1)], csem
        )

    for t in range(TPW):
        ccopy(t).start()
    mcopy = pltpu.make_async_copy(
        mv.at[:, pl.ds(bt0 * BT, TPW * BT)], mbuf, msem
    )
    mcopy.start()
    mcopy.wait()

    zero = jnp.zeros((16,), jnp.float32)

    def btstep(t, accs):
        ccopy(t).wait()

        def edge(e, xs, ys, nums, wss):
            for s in range(S):
                xn = cbuf[e + 1, t, 0, pl.ds(16 * s, 16)]
                yn = cbuf[e + 1, t, 1, pl.ds(16 * s, 16)]
                w = mbuf[e, pl.ds(t * BT + 16 * s, 16)]
                d = jnp.abs(xs[s] - xn) + jnp.abs(ys[s] - yn)
                nums[s] = nums[s] + d * w
                wss[s] = wss[s] + w
                xs[s] = xn
                ys[s] = yn

        xs = [cbuf[0, t, 0, pl.ds(16 * s, 16)] for s in range(S)]
        ys = [cbuf[0, t, 1, pl.ds(16 * s, 16)] for s in range(S)]
        nums = [zero] * S
        wss = [zero] * S

        def estep(g, carry):
            xs, ys, nums, wss = map(list, carry)
            for u in range(UNROLL):
                edge(g * UNROLL + u, xs, ys, nums, wss)
            return tuple(xs), tuple(ys), tuple(nums), tuple(wss)

        carry = (tuple(xs), tuple(ys), tuple(nums), tuple(wss))
        xs, ys, nums, wss = map(list, lax.fori_loop(0, NLOOP, estep, carry))
        for e in range(NLOOP * UNROLL, E):
            edge(e, xs, ys, nums, wss)
        return tuple(
            accs[s] + nums[s] / jnp.maximum(wss[s], 1.0) for s in range(S)
        )

    accs = lax.fori_loop(0, TPW, btstep, (zero,) * S)
    total = accs[0]
    for s in range(1, S):
        total = total + accs[s]
    accbuf[...] = total
    pltpu.sync_copy(accbuf, out_hbm.at[wid])


def kernel(coords, mask_edges):
    cv = coords.reshape(NBT, BT, K, 2).transpose(2, 0, 3, 1)
    mv = mask_edges.transpose(1, 0)
    mesh = plsc.VectorSubcoreMesh(core_axis_name="c", subcore_axis_name="s")
    k = functools.partial(
        pl.kernel,
        mesh=mesh,
        compiler_params=pltpu.CompilerParams(needs_layout_passes=False),
        out_type=jax.ShapeDtypeStruct((NW, 16), jnp.float32),
        scratch_types=[
            pltpu.VMEM((K, TPW, 2, BT), jnp.float32),
            pltpu.VMEM((E, TPW * BT), jnp.float32),
            pltpu.VMEM((16,), jnp.float32),
            pltpu.SemaphoreType.DMA,
            pltpu.SemaphoreType.DMA,
        ],
    )(_sc_body)
    partials = k(cv, mv)
    return (WEIGHT / B) * jnp.sum(partials)


# speedup vs baseline: 1.1520x; 1.1520x over previous
import functools

import jax
import jax.numpy as jnp
from jax import lax
from jax.experimental import pallas as pl
from jax.experimental.pallas import tpu as pltpu
from jax.experimental.pallas import tpu_sc as plsc

B = 16384
K = 68
E = 67
NC = 2
NS = 16
NW = NC * NS
BT = 128
NBT = B // BT
S = BT // 16
NSC = 96
TPW = NSC // NW
NB_TC = 16
GRID = (NBT - NSC) // NB_TC
WEIGHT = 0.05


def _sc_body(cv, mv, out_hbm, cbuf0, cbuf1, mbuf0, mbuf1, accbuf, sem0, sem1):
    wid = lax.axis_index("s") * NC + lax.axis_index("c")
    bt0 = wid * TPW
    cbufs = (cbuf0, cbuf1)
    mbufs = (mbuf0, mbuf1)
    sems = (sem0, sem1)

    def ccopy(t, slot):
        return pltpu.make_async_copy(
            cv.at[:, pl.ds(bt0 + t, 1)], cbufs[slot], sems[slot]
        )

    def mcopy(t, slot):
        return pltpu.make_async_copy(
            mv.at[:, pl.ds((bt0 + t) * BT, BT)], mbufs[slot], sems[slot]
        )

    ccopy(0, 0).start()
    mcopy(0, 0).start()

    zero = jnp.zeros((16,), jnp.float32)
    accs = [zero] * S
    for t in range(TPW):
        slot = t % 2
        if t + 1 < TPW:
            ccopy(t + 1, 1 - slot).start()
            mcopy(t + 1, 1 - slot).start()
        ccopy(t, slot).wait()
        mcopy(t, slot).wait()
        cb = cbufs[slot]
        mb = mbufs[slot]

        xs = [cb[0, 0, 0, pl.ds(16 * s, 16)] for s in range(S)]
        ys = [cb[0, 0, 1, pl.ds(16 * s, 16)] for s in range(S)]
        nums = [zero] * S
        wss = [zero] * S

        def estep(e, carry, cb=cb, mb=mb):
            xs, ys, nums, wss = map(list, carry)
            for s in range(S):
                xn = cb[e + 1, 0, 0, pl.ds(16 * s, 16)]
                yn = cb[e + 1, 0, 1, pl.ds(16 * s, 16)]
                w = mb[e, pl.ds(16 * s, 16)]
                d = jnp.abs(xs[s] - xn) + jnp.abs(ys[s] - yn)
                nums[s] = nums[s] + d * w
                wss[s] = wss[s] + w
                xs[s] = xn
                ys[s] = yn
            return tuple(xs), tuple(ys), tuple(nums), tuple(wss)

        carry = (tuple(xs), tuple(ys), tuple(nums), tuple(wss))
        _, _, nums, wss = lax.fori_loop(0, E, estep, carry)
        for s in range(S):
            accs[s] = accs[s] + nums[s] / jnp.maximum(wss[s], 1.0)

    total = accs[0]
    for s in range(1, S):
        total = total + accs[s]
    accbuf[...] = total
    pltpu.sync_copy(accbuf, out_hbm.at[wid])


def _tc_body(cref, mref, oref):
    i = pl.program_id(0)

    @pl.when(i == 0)
    def _init():
        oref[...] = jnp.zeros_like(oref)

    acc = jnp.zeros((1, BT), jnp.float32)
    for j in range(NB_TC):
        x = cref[:, j, 0, :]
        y = cref[:, j, 1, :]
        d = jnp.abs(x[:-1, :] - x[1:, :]) + jnp.abs(y[:-1, :] - y[1:, :])
        w = mref[:, j * BT:(j + 1) * BT]
        num = jnp.sum(d * w, axis=0, keepdims=True)
        ws = jnp.sum(w, axis=0, keepdims=True)
        acc = acc + num / jnp.maximum(ws, 1.0)
    oref[...] += acc


def kernel(coords, mask_edges):
    cv = coords.reshape(NBT, BT, K, 2).transpose(2, 0, 3, 1)
    mv = mask_edges.transpose(1, 0)

    mesh = plsc.VectorSubcoreMesh(core_axis_name="c", subcore_axis_name="s")
    sc_k = functools.partial(
        pl.kernel,
        mesh=mesh,
        compiler_params=pltpu.CompilerParams(needs_layout_passes=False),
        out_type=jax.ShapeDtypeStruct((NW, 16), jnp.float32),
        scratch_types=[
            pltpu.VMEM((K, 1, 2, BT), jnp.float32),
            pltpu.VMEM((K, 1, 2, BT), jnp.float32),
            pltpu.VMEM((E, BT), jnp.float32),
            pltpu.VMEM((E, BT), jnp.float32),
            pltpu.VMEM((16,), jnp.float32),
            pltpu.SemaphoreType.DMA,
            pltpu.SemaphoreType.DMA,
        ],
    )(_sc_body)
    sc_partials = sc_k(cv, mv)

    tc_partial = pl.pallas_call(
        _tc_body,
        grid=(GRID,),
        in_specs=[
            pl.BlockSpec((K, NB_TC, 2, BT), lambda i: (0, NSC // NB_TC + i, 0, 0)),
            pl.BlockSpec((E, NB_TC * BT), lambda i: (0, NSC // NB_TC + i)),
        ],
        out_specs=pl.BlockSpec((1, BT), lambda i: (0, 0)),
        out_shape=jax.ShapeDtypeStruct((1, BT), jnp.float32),
    )(cv, mv)

    return (WEIGHT / B) * (jnp.sum(sc_partials) + jnp.sum(tc_partial))
